# baseline (speedup 1.0000x reference)
import jax
import jax.numpy as jnp
from jax.experimental import pallas as pl
from jax.experimental.pallas import tpu as pltpu

_BS = 1024
_U = 8


def _scan_body(x_ref, o_ref, carry_ref):
    j = pl.program_id(1)

    @pl.when(j == 0)
    def _():
        carry_ref[...] = jnp.zeros_like(carry_ref)

    L = x_ref.shape[2]

    def group(k, carry):
        vs = []
        for u in range(_U):
            v = x_ref[0, pl.ds(k * (8 * _U) + u * 8, 8), :]
            for d in (1, 2, 4):
                v = v + jnp.concatenate(
                    [jnp.zeros((d, L), v.dtype), v[: 8 - d]], axis=0
                )
            vs.append(v)
        offs = [carry]
        for u in range(_U - 1):
            offs.append(offs[-1] + vs[u][7:8, :])
        for u in range(_U):
            o_ref[0, pl.ds(k * (8 * _U) + u * 8, 8), :] = vs[u] + offs[u]
        return offs[_U - 1] + vs[_U - 1][7:8, :]

    carry_ref[...] = jax.lax.fori_loop(0, _BS // (8 * _U), group, carry_ref[...])


def kernel(x):
    B, S, L = x.shape
    grid = (B, S // _BS)
    return pl.pallas_call(
        _scan_body,
        grid=grid,
        in_specs=[pl.BlockSpec((1, _BS, L), lambda i, j: (i, j, 0))],
        out_specs=pl.BlockSpec((1, _BS, L), lambda i, j: (i, j, 0)),
        out_shape=jax.ShapeDtypeStruct(x.shape, x.dtype),
        scratch_shapes=[pltpu.VMEM((1, L), jnp.float32)],
        compiler_params=pltpu.CompilerParams(
            dimension_semantics=("arbitrary", "arbitrary"),
        ),
    )(x)


# speedup vs baseline: 1.0181x; 1.0181x over previous
import jax
import jax.numpy as jnp
from jax.experimental import pallas as pl
from jax.experimental.pallas import tpu as pltpu

_BS = 512
_U = 4
_NB = 2


def _scan_body(x_ref, o_ref, carry_ref):
    j = pl.program_id(1)

    @pl.when(j == 0)
    def _():
        carry_ref[...] = jnp.zeros_like(carry_ref)

    L = x_ref.shape[2]
    rows = 8 * _U

    def group(k, carry):
        newc = []
        for n in range(_NB):
            vs = []
            for u in range(_U):
                v = x_ref[n, pl.ds(k * rows + u * 8, 8), :]
                for d in (1, 2, 4):
                    v = v + jnp.concatenate(
                        [jnp.zeros((d, L), v.dtype), v[: 8 - d]], axis=0
                    )
                vs.append(v)
            offs = [carry[n : n + 1]]
            for u in range(_U - 1):
                offs.append(offs[-1] + vs[u][7:8, :])
            for u in range(_U):
                o_ref[n, pl.ds(k * rows + u * 8, 8), :] = vs[u] + offs[u]
            newc.append(offs[_U - 1] + vs[_U - 1][7:8, :])
        return jnp.concatenate(newc, axis=0)

    carry_ref[...] = jax.lax.fori_loop(0, _BS // rows, group, carry_ref[...])


def kernel(x):
    B, S, L = x.shape
    grid = (B // _NB, S // _BS)
    return pl.pallas_call(
        _scan_body,
        grid=grid,
        in_specs=[pl.BlockSpec((_NB, _BS, L), lambda i, j: (i, j, 0))],
        out_specs=pl.BlockSpec((_NB, _BS, L), lambda i, j: (i, j, 0)),
        out_shape=jax.ShapeDtypeStruct(x.shape, x.dtype),
        scratch_shapes=[pltpu.VMEM((_NB, L), jnp.float32)],
        compiler_params=pltpu.CompilerParams(
            dimension_semantics=("arbitrary", "arbitrary"),
        ),
    )(x)


# speedup vs baseline: 1.0188x; 1.0007x over previous
import jax
import jax.numpy as jnp
from jax.experimental import pallas as pl
from jax.experimental.pallas import tpu as pltpu

_BS = 256
_U = 4
_NB = 4


def _scan_body(x_ref, o_ref, carry_ref):
    j = pl.program_id(1)

    @pl.when(j == 0)
    def _():
        carry_ref[...] = jnp.zeros_like(carry_ref)

    L = x_ref.shape[2]
    rows = 8 * _U

    def group(k, carry):
        newc = []
        for n in range(_NB):
            vs = []
            for u in range(_U):
                v = x_ref[n, pl.ds(k * rows + u * 8, 8), :]
                for d in (1, 2, 4):
                    v = v + jnp.concatenate(
                        [jnp.zeros((d, L), v.dtype), v[: 8 - d]], axis=0
                    )
                vs.append(v)
            offs = [carry[n : n + 1]]
            for u in range(_U - 1):
                offs.append(offs[-1] + vs[u][7:8, :])
            for u in range(_U):
                o_ref[n, pl.ds(k * rows + u * 8, 8), :] = vs[u] + offs[u]
            newc.append(offs[_U - 1] + vs[_U - 1][7:8, :])
        return jnp.concatenate(newc, axis=0)

    carry_ref[...] = jax.lax.fori_loop(0, _BS // rows, group, carry_ref[...])


def kernel(x):
    B, S, L = x.shape
    grid = (B // _NB, S // _BS)
    return pl.pallas_call(
        _scan_body,
        grid=grid,
        in_specs=[pl.BlockSpec((_NB, _BS, L), lambda i, j: (i, j, 0))],
        out_specs=pl.BlockSpec((_NB, _BS, L), lambda i, j: (i, j, 0)),
        out_shape=jax.ShapeDtypeStruct(x.shape, x.dtype),
        scratch_shapes=[pltpu.VMEM((_NB, L), jnp.float32)],
        compiler_params=pltpu.CompilerParams(
            dimension_semantics=("arbitrary", "arbitrary"),
        ),
    )(x)
